# baseline (speedup 1.0000x reference)
import jax
import jax.numpy as jnp
from jax.experimental import pallas as pl
from jax.experimental.pallas import tpu as pltpu

_B = 1024
_K = 100000
_O = 64
_KBLK = 2048
_NSTEP = (_K + _KBLK - 1) // _KBLK
_DIMNUMS = (((1,), (1,)), ((), ()))


def _mm_body(x_ref, w_ref, b_ref, o_ref):
    k = pl.program_id(0)

    @pl.when(k == 0)
    def _init():
        o_ref[...] = jnp.broadcast_to(b_ref[...], (_B, _O))

    @pl.when(k < _NSTEP - 1)
    def _full():
        o_ref[...] += jax.lax.dot_general(
            x_ref[...], w_ref[...], _DIMNUMS,
            preferred_element_type=jnp.float32)

    @pl.when(k == _NSTEP - 1)
    def _tail():
        valid = _K - (_NSTEP - 1) * _KBLK
        x = jnp.where(
            jax.lax.broadcasted_iota(jnp.int32, (_B, _KBLK), 1) < valid,
            x_ref[...], 0.0)
        w = jnp.where(
            jax.lax.broadcasted_iota(jnp.int32, (_O, _KBLK), 1) < valid,
            w_ref[...], 0.0)
        o_ref[...] += jax.lax.dot_general(
            x, w, _DIMNUMS, preferred_element_type=jnp.float32)


def kernel(input, weight, bias):
    out = pl.pallas_call(
        _mm_body,
        grid=(_NSTEP,),
        in_specs=[
            pl.BlockSpec((_B, _KBLK), lambda k: (0, k)),
            pl.BlockSpec((_O, _KBLK), lambda k: (0, k)),
            pl.BlockSpec((1, _O), lambda k: (0, 0)),
        ],
        out_specs=pl.BlockSpec((_B, _O), lambda k: (0, 0)),
        out_shape=jax.ShapeDtypeStruct((_B, _O), jnp.float32),
        compiler_params=pltpu.CompilerParams(
            dimension_semantics=("arbitrary",),
        ),
    )(input, weight, bias.reshape(1, _O))
    return out


# speedup vs baseline: 3.7515x; 3.7515x over previous
import jax
import jax.numpy as jnp
from jax.experimental import pallas as pl
from jax.experimental.pallas import tpu as pltpu

_B = 1024
_K = 100000
_O = 64
_KBLK = 2048
_NSTEP = (_K + _KBLK - 1) // _KBLK
_DIMNUMS = (((1,), (0,)), ((), ()))


def _mm_body(x_ref, w_ref, b_ref, o_ref):
    k = pl.program_id(0)

    @pl.when(k == 0)
    def _init():
        o_ref[...] = jnp.broadcast_to(b_ref[...], (_O, _B))

    @pl.when(k < _NSTEP - 1)
    def _full():
        o_ref[...] += jax.lax.dot_general(
            w_ref[...], x_ref[...], _DIMNUMS,
            preferred_element_type=jnp.float32)

    @pl.when(k == _NSTEP - 1)
    def _tail():
        valid = _K - (_NSTEP - 1) * _KBLK
        x = jnp.where(
            jax.lax.broadcasted_iota(jnp.int32, (_KBLK, _B), 0) < valid,
            x_ref[...], 0.0)
        w = jnp.where(
            jax.lax.broadcasted_iota(jnp.int32, (_O, _KBLK), 1) < valid,
            w_ref[...], 0.0)
        o_ref[...] += jax.lax.dot_general(
            w, x, _DIMNUMS, preferred_element_type=jnp.float32)


def kernel(input, weight, bias):
    out_t = pl.pallas_call(
        _mm_body,
        grid=(_NSTEP,),
        in_specs=[
            pl.BlockSpec((_KBLK, _B), lambda k: (k, 0)),
            pl.BlockSpec((_O, _KBLK), lambda k: (0, k)),
            pl.BlockSpec((_O, 1), lambda k: (0, 0)),
        ],
        out_specs=pl.BlockSpec((_O, _B), lambda k: (0, 0)),
        out_shape=jax.ShapeDtypeStruct((_O, _B), jnp.float32),
        compiler_params=pltpu.CompilerParams(
            dimension_semantics=("arbitrary",),
        ),
    )(input.T, weight, bias.reshape(_O, 1))
    return out_t.T
